# baseline (speedup 1.0000x reference)
import functools
import math

import jax
import jax.numpy as jnp
from jax import lax
from jax.experimental import pallas as pl
from jax.experimental.pallas import tpu as pltpu
from jax.experimental.pallas import tpu_sc as plsc

EMBED_DIM = 64
SCALE = math.sqrt(EMBED_DIM)

B, T = 4096, 200
N = B * T
VOCAB = 1000000
NUM_CORES = 2
NUM_SUBCORES = 16
NW = NUM_CORES * NUM_SUBCORES

NVB = VOCAB // 128
K1_STEPS = 245

K2_UNITS_PER_W = (25 * 32) // NW


def _iota16():
    return lax.iota(jnp.int32, 16)


def _k1_transpose(table_t, tail64):
    mesh = plsc.VectorSubcoreMesh(core_axis_name="c", subcore_axis_name="s")

    scratch = (
        [pltpu.VMEM((64, 128), jnp.float32)] * 2
        + [pltpu.VMEM((64, 128), jnp.float32)]
        + [pltpu.VMEM((64, 64), jnp.float32)]
        + [pltpu.VMEM((32, 128), jnp.float32)]
        + [pltpu.SemaphoreType.DMA] * 2
    )

    @functools.partial(
        pl.kernel,
        mesh=mesh,
        compiler_params=pltpu.CompilerParams(needs_layout_passes=False),
        out_type=jax.ShapeDtypeStruct((VOCAB // 2, 128), jnp.float32),
        scratch_types=scratch,
    )
    def k1(tt_hbm, tail_hbm, t2_hbm, buf0, buf1, stage, tbuf, tstage, sem0, sem1):
        wid = lax.axis_index("s") * NUM_CORES + lax.axis_index("c")
        bufs = (buf0, buf1)
        sems = (sem0, sem1)

        def blkof(k):
            return jnp.minimum(wid + NW * k, NVB - 1)

        def fire(bi, vb):
            off = pl.multiple_of(vb * 128, 128)
            pltpu.async_copy(tt_hbm.at[:, pl.ds(off, 128)], bufs[bi], sems[bi])

        def wait(bi, vb):
            off = pl.multiple_of(vb * 128, 128)
            pltpu.make_async_copy(
                tt_hbm.at[:, pl.ds(off, 128)], bufs[bi], sems[bi]
            ).wait()

        def process(bi, vb):
            buf = bufs[bi]

            def prow(p, _):
                p8 = pl.multiple_of(p * 8, 8)
                for kk in range(8):
                    q_in = p8 + kk
                    for h in range(2):
                        col = 2 * q_in + h
                        for jg in range(4):
                            stage[q_in, pl.ds(h * 64 + jg * 16, 16)] = (
                                plsc.load_gather(
                                    buf,
                                    [
                                        jg * 16 + _iota16(),
                                        jnp.full((16,), 0, jnp.int32) + col,
                                    ],
                                )
                                * SCALE
                            )
                return 0

            lax.fori_loop(0, 8, prow, 0)
            q0 = pl.multiple_of(vb * 64, 64)
            pltpu.sync_copy(stage, t2_hbm.at[pl.ds(q0, 64)])

        fire(0, blkof(0))

        def pair_body(i2, _):
            k0 = 2 * i2
            wait(0, blkof(k0))
            fire(1, blkof(k0 + 1))
            process(0, blkof(k0))
            wait(1, blkof(k0 + 1))
            fire(0, blkof(k0 + 2))
            process(1, blkof(k0 + 1))
            return 0

        lax.fori_loop(0, (K1_STEPS - 1) // 2, pair_body, 0)
        wait(0, blkof(K1_STEPS - 1))
        process(0, blkof(K1_STEPS - 1))

        @pl.when(wid == NW - 1)
        def _():
            pltpu.sync_copy(tail_hbm, tbuf)

            def trow(p, _):
                p8 = pl.multiple_of(p * 8, 8)
                for kk in range(8):
                    q_in = p8 + kk
                    for h in range(2):
                        col = 2 * q_in + h
                        for jg in range(4):
                            tstage[q_in, pl.ds(h * 64 + jg * 16, 16)] = (
                                plsc.load_gather(
                                    tbuf,
                                    [
                                        jnp.full((16,), 0, jnp.int32) + col,
                                        jg * 16 + _iota16(),
                                    ],
                                )
                                * SCALE
                            )
                return 0

            lax.fori_loop(0, 4, trow, 0)
            pltpu.sync_copy(tstage, t2_hbm.at[pl.ds((VOCAB - 64) // 2, 32)])

    return k1(table_t, tail64)


def _k2_gather(x5, t2):
    mesh = plsc.VectorSubcoreMesh(core_axis_name="c", subcore_axis_name="s")

    scratch = (
        [pltpu.VMEM((8, 128), jnp.int32)]
        + [pltpu.VMEM((8, 128), jnp.int32)]
        + [pltpu.VMEM((8, 128), jnp.int32)]
        + [pltpu.VMEM((128, 128), jnp.float32)] * 2
        + [pltpu.VMEM((8, 8, 128), jnp.float32)]
        + [pltpu.SemaphoreType.DMA] * 2
    )

    @functools.partial(
        pl.kernel,
        mesh=mesh,
        compiler_params=pltpu.CompilerParams(needs_layout_passes=False),
        out_type=jax.ShapeDtypeStruct((200, 8, 32, 8, 128), jnp.float32),
        scratch_types=scratch,
    )
    def k2(x5_hbm, t2_hbm, out_hbm, idxb, idx2b, pvb, rows0, rows1, stage,
           sem0, sem1):
        wid = lax.axis_index("s") * NUM_CORES + lax.axis_index("c")
        rows = (rows0, rows1)
        sems = (sem0, sem1)

        def fire(bi, ti):
            pltpu.async_copy(t2_hbm.at[idx2b.at[ti]], rows[bi], sems[bi])

        def wait(bi, ti):
            pltpu.make_async_copy(
                t2_hbm.at[idx2b.at[ti]], rows[bi], sems[bi]
            ).wait()

        def unit_body(u, _):
            uid = wid * K2_UNITS_PER_W + u
            tb = uid // 32
            bb = uid % 32
            pltpu.sync_copy(x5_hbm.at[tb, bb], idxb)
            for ti in range(8):
                for g in range(8):
                    sl = pl.ds(g * 16, 16)
                    raw = idxb[ti, sl]
                    idx2b[ti, sl] = lax.shift_right_logical(raw, 1)
                    pvb[ti, sl] = (raw & 1) * 64
            fire(0, 0)
            for ti in range(8):
                bi = ti % 2
                wait(bi, ti)
                if ti < 7:
                    fire(1 - bi, ti + 1)

                def gbody(g, _, ti=ti, bi=bi):
                    lanes = g * 16 + _iota16()
                    tisplat = jnp.full((16,), ti, jnp.int32)
                    pv = plsc.load_gather(pvb, [tisplat, lanes])
                    for jb in range(8):
                        for ji in range(8):
                            v = plsc.load_gather(
                                rows[bi], [lanes, pv + (jb * 8 + ji)]
                            )
                            plsc.store_scatter(
                                stage,
                                [
                                    jnp.full((16,), jb, jnp.int32),
                                    jnp.full((16,), ji, jnp.int32),
                                    lanes,
                                ],
                                v,
                            )
                    return 0

                lax.fori_loop(0, 8, gbody, 0)
                t = tb * 8 + ti
                pltpu.sync_copy(stage, out_hbm.at[t, :, bb])
            return 0

        lax.fori_loop(0, K2_UNITS_PER_W, unit_body, 0)

    return k2(x5, t2)


def kernel(x, table):
    x5 = x.T.reshape(25, 8, 32, 128).transpose(0, 2, 1, 3)
    tail64 = table[VOCAB - 64:, :]
    t2 = _k1_transpose(table.T, tail64)
    out5 = _k2_gather(x5, t2)
    return out5.transpose(2, 4, 0, 1, 3).reshape(B, T, EMBED_DIM)


# speedup vs baseline: 2.2176x; 2.2176x over previous
import functools
import math

import jax
import jax.numpy as jnp
from jax import lax
from jax.experimental import pallas as pl
from jax.experimental.pallas import tpu as pltpu
from jax.experimental.pallas import tpu_sc as plsc

EMBED_DIM = 64
SCALE = math.sqrt(EMBED_DIM)

B, T = 4096, 200
N = B * T
NUM_CORES = 2
NUM_SUBCORES = 16
NW = NUM_CORES * NUM_SUBCORES
XROWS_PER_W = B // NW
CA, CB = 104, 96


def _sc_embedding_lookup(x_flat, table):
    mesh = plsc.VectorSubcoreMesh(core_axis_name="c", subcore_axis_name="s")

    scratch = (
        [pltpu.VMEM((CA,), jnp.int32)] * 2
        + [pltpu.VMEM((CB,), jnp.int32)] * 2
        + [pltpu.VMEM((CA, EMBED_DIM), jnp.float32)] * 2
        + [pltpu.VMEM((CB, EMBED_DIM), jnp.float32)] * 2
        + [pltpu.SemaphoreType.DMA] * 4
    )

    @functools.partial(
        pl.kernel,
        mesh=mesh,
        compiler_params=pltpu.CompilerParams(use_tc_tiling_on_sc=False),
        out_type=jax.ShapeDtypeStruct((B, T, EMBED_DIM), jnp.float32),
        scratch_types=scratch,
    )
    def k(idx_hbm, table_hbm, out_hbm, *sc):
        idxa = sc[0:2]
        idxb = sc[2:4]
        rowsa = sc[4:6]
        rowsb = sc[6:8]
        sems = sc[8:12]
        wid = lax.axis_index("s") * NUM_CORES + lax.axis_index("c")
        base = wid * XROWS_PER_W

        def fire(xr, rr, half):
            idx = (idxa, idxb)[half][rr]
            rows = (rowsa, rowsb)[half][rr]
            sem = sems[half * 2 + rr]
            cn = (CA, CB)[half]
            off = pl.multiple_of((base + xr) * T + half * CA, 8)
            pltpu.sync_copy(idx_hbm.at[pl.ds(off, cn)], idx)
            pltpu.async_copy(table_hbm.at[idx], rows, sem)

        def finish(xr, rr, half):
            idx = (idxa, idxb)[half][rr]
            rows = (rowsa, rowsb)[half][rr]
            sem = sems[half * 2 + rr]
            cn = (CA, CB)[half]
            pltpu.make_async_copy(table_hbm.at[idx], rows, sem).wait()

            def scale_body(i, _):
                for j in range(EMBED_DIM // 16):
                    sl = pl.ds(j * 16, 16)
                    rows[i, sl] = rows[i, sl] * SCALE
                return 0

            lax.fori_loop(0, cn, scale_body, 0)
            dst = out_hbm.at[base + xr].at[pl.ds(half * CA, cn)]
            pltpu.sync_copy(rows, dst)

        for r in range(2):
            fire(r, r, 0)
            fire(r, r, 1)

        def pair_body(g, _):
            for rr in range(2):
                xr = g * 2 + rr
                for half in range(2):
                    finish(xr, rr, half)
                    fire(xr + 2, rr, half)
            return 0

        lax.fori_loop(0, (XROWS_PER_W - 2) // 2, pair_body, 0)

        for rr in range(2):
            xr = XROWS_PER_W - 2 + rr
            for half in range(2):
                finish(xr, rr, half)

    return k(x_flat, table)


def kernel(x, table):
    x_flat = x.reshape(N)
    return _sc_embedding_lookup(x_flat, table)


# speedup vs baseline: 2.5349x; 1.1431x over previous
import functools
import math

import jax
import jax.numpy as jnp
from jax import lax
from jax.experimental import pallas as pl
from jax.experimental.pallas import tpu as pltpu
from jax.experimental.pallas import tpu_sc as plsc

EMBED_DIM = 64
SCALE = math.sqrt(EMBED_DIM)

B, T = 4096, 200
N = B * T
NUM_CORES = 2
NUM_SUBCORES = 16
NW = NUM_CORES * NUM_SUBCORES
ROWS_PER_W = N // NW
CHUNK = 128
NCHUNK = ROWS_PER_W // CHUNK
NBUF = 8
NGROUP = NCHUNK // NBUF


def _sc_embedding_lookup(x_flat, table):
    mesh = plsc.VectorSubcoreMesh(core_axis_name="c", subcore_axis_name="s")

    scratch = (
        [pltpu.VMEM((ROWS_PER_W,), jnp.int32)]
        + [pltpu.VMEM((CHUNK, EMBED_DIM), jnp.float32)] * NBUF
        + [pltpu.SemaphoreType.DMA] * NBUF
    )

    @functools.partial(
        pl.kernel,
        mesh=mesh,
        compiler_params=pltpu.CompilerParams(use_tc_tiling_on_sc=False),
        out_type=jax.ShapeDtypeStruct((N, EMBED_DIM), jnp.float32),
        scratch_types=scratch,
    )
    def k(idx_hbm, table_hbm, out_hbm, idx_v, *bufs_and_sems):
        rows = bufs_and_sems[:NBUF]
        sems = bufs_and_sems[NBUF:]
        wid = lax.axis_index("s") * NUM_CORES + lax.axis_index("c")
        base = wid * ROWS_PER_W

        pltpu.sync_copy(idx_hbm.at[pl.ds(base, ROWS_PER_W)], idx_v)

        def fire(ci, b):
            src = table_hbm.at[idx_v.at[pl.ds(ci * CHUNK, CHUNK)]]
            pltpu.async_copy(src, rows[b], sems[b])

        def drain(ci, b):
            src = table_hbm.at[idx_v.at[pl.ds(ci * CHUNK, CHUNK)]]
            pltpu.make_async_copy(src, rows[b], sems[b]).wait()

        def scale_and_store(ci, b):
            @plsc.parallel_loop(0, CHUNK, step=1, unroll=4)
            def _(r):
                for j in range(EMBED_DIM // 16):
                    sl = pl.ds(j * 16, 16)
                    rows[b][r, sl] = rows[b][r, sl] * SCALE

            pltpu.sync_copy(rows[b], out_hbm.at[pl.ds(base + ci * CHUNK, CHUNK)])

        for b in range(NBUF):
            fire(b, b)

        def group_body(g, _):
            for b in range(NBUF):
                ci = g * NBUF + b
                drain(ci, b)
                scale_and_store(ci, b)
                fire(ci + NBUF, b)
            return 0

        lax.fori_loop(0, NGROUP - 1, group_body, 0)

        for b in range(NBUF):
            ci = (NGROUP - 1) * NBUF + b
            drain(ci, b)
            scale_and_store(ci, b)

    return k(x_flat, table)


def kernel(x, table):
    x_flat = x.reshape(N)
    out = _sc_embedding_lookup(x_flat, table)
    return out.reshape(B, T, EMBED_DIM)
